# Initial kernel scaffold; baseline (speedup 1.0000x reference)
import jax
import jax.numpy as jnp
from jax.experimental import pallas as pl


def kernel(image, feature_map, target, conv_w, conv_b, cls_w, cls_b, box_w, box_b):
    raise NotImplementedError("write your pallas kernel here")


# speedup vs baseline: 20.9783x; 20.9783x over previous
import functools

import jax
import jax.numpy as jnp
import numpy as np
from jax import lax
from jax.experimental import pallas as pl
from jax.experimental.pallas import tpu as pltpu

_SCALES = (128.0, 256.0, 512.0)
_ASPECT = (0.5, 1.0, 2.0)
_PRE = 10000
_POST = 2000
_THR = 0.7

_N = 16650
_NP = 16896
_NS = 10240
_NB = 80
_RB = 2112
_RC = 1408


def _anchors(image_shape, feat_shape):
    grid_h, grid_w = feat_shape[-2], feat_shape[-1]
    image_h, image_w = image_shape[-2], image_shape[-1]
    stride_h = image_h // grid_h
    stride_w = image_w // grid_w
    scales = jnp.asarray(_SCALES, jnp.float32)
    ar = jnp.asarray(_ASPECT, jnp.float32)
    h = jnp.sqrt(ar)
    w = 1.0 / h
    ws = (w[:, None] * scales[None, :]).reshape(-1)
    hs = (h[:, None] * scales[None, :]).reshape(-1)
    base = jnp.round(jnp.stack([-ws, -hs, ws, hs], axis=1) / 2.0)
    sx = jnp.arange(0, grid_w, dtype=jnp.float32) * stride_w
    sy = jnp.arange(0, grid_h, dtype=jnp.float32) * stride_h
    syy, sxx = jnp.meshgrid(sy, sx, indexing="ij")
    sxx = sxx.reshape(-1)
    syy = syy.reshape(-1)
    shifts = jnp.stack([sxx, syy, sxx, syy], axis=1)
    return (shifts[:, None, :] + base[None, :, :]).reshape(-1, 4)


def _prep_kernel(img_h, img_w, logit_ref, dlt_ref, anc_ref, out_ref):
    score = jax.nn.sigmoid(logit_ref[0:1, :])
    a0 = anc_ref[0:1, :]
    a1 = anc_ref[1:2, :]
    a2 = anc_ref[2:3, :]
    a3 = anc_ref[3:4, :]
    w = a2 - a0
    h = a3 - a1
    cx = a0 + 0.5 * w
    cy = a1 + 0.5 * h
    tx = dlt_ref[0:1, :]
    ty = dlt_ref[1:2, :]
    tw = dlt_ref[2:3, :]
    th = dlt_ref[3:4, :]
    pcx = tx * w + cx
    pcy = ty * h + cy
    pw = jnp.exp(tw) * w
    ph = jnp.exp(th) * h
    x1 = pcx - 0.5 * pw
    y1 = pcy - 0.5 * ph
    x2 = pcx + 0.5 * pw
    y2 = pcy + 0.5 * ph
    out_ref[0:1, :] = score
    out_ref[1:2, :] = jnp.clip(x1, 0.0, img_w)
    out_ref[2:3, :] = jnp.clip(y1, 0.0, img_h)
    out_ref[3:4, :] = jnp.clip(x2, 0.0, img_w)
    out_ref[4:5, :] = jnp.clip(y2, 0.0, img_h)
    out_ref[5:6, :] = jnp.zeros((1, _NP), jnp.float32)
    out_ref[6:7, :] = jnp.zeros((1, _NP), jnp.float32)
    out_ref[7:8, :] = jnp.zeros((1, _NP), jnp.float32)


def _rank_kernel(scol_ref, srow_ref, rank_ref):
    pid = pl.program_id(0)
    s_col = scol_ref[...]
    i_col = pid * _RB + lax.broadcasted_iota(jnp.int32, (_RB, 1), 0)

    def body(c, acc):
        s_ch = srow_ref[0:1, pl.ds(c * _RC, _RC)]
        i_ch = c * _RC + lax.broadcasted_iota(jnp.int32, (1, _RC), 1)
        cmp = (s_ch > s_col) | ((s_ch == s_col) & (i_ch < i_col))
        return acc + jnp.sum(cmp.astype(jnp.int32), axis=1, keepdims=True)

    rank_ref[...] = lax.fori_loop(0, _NP // _RC, body, jnp.zeros((_RB, 1), jnp.int32))


def _scatter_kernel(rank_ref, vals_ref, out_ref):
    pid = pl.program_id(0)
    r_row = pid * 512 + lax.broadcasted_iota(jnp.int32, (1, 512), 1)

    def body(c, acc):
        rk = rank_ref[pl.ds(c * 4224, 4224), :]
        p = (rk == r_row).astype(jnp.float32)
        v = vals_ref[:, pl.ds(c * 4224, 4224)]
        return acc + lax.dot_general(v, p, (((1,), (0,)), ((), ())),
                                     preferred_element_type=jnp.float32,
                                     precision="highest")

    out_ref[...] = lax.fori_loop(0, 4, body, jnp.zeros((8, 512), jnp.float32))


def _nms_kernel(sorted_ref, dest_ref, masked_ref, m_ref, keep_ref):
    masked_ref[0:4, :] = jnp.full((4, _NS), -1e9, jnp.float32)
    masked_ref[4:5, :] = jnp.zeros((1, _NS), jnp.float32)
    lane = lax.broadcasted_iota(jnp.int32, (1, 128), 1)

    def block(b, _):
        off = b * 128
        x1r = sorted_ref[1:2, pl.ds(off, 128)]
        y1r = sorted_ref[2:3, pl.ds(off, 128)]
        x2r = sorted_ref[3:4, pl.ds(off, 128)]
        y2r = sorted_ref[4:5, pl.ds(off, 128)]
        area_r = (x2r - x1r) * (y2r - y1r)

        def colm(v):
            return jnp.transpose(jnp.broadcast_to(v, (128, 128)))

        x1m = colm(x1r)
        y1m = colm(y1r)
        x2m = colm(x2r)
        y2m = colm(y2r)
        am = colm(area_r)
        x1c = x1m[:, 0:1]
        y1c = y1m[:, 0:1]
        x2c = x2m[:, 0:1]
        y2c = y2m[:, 0:1]
        ac = am[:, 0:1]

        def chunk(p, sup):
            mx1 = masked_ref[0:1, pl.ds(p * 2048, 2048)]
            my1 = masked_ref[1:2, pl.ds(p * 2048, 2048)]
            mx2 = masked_ref[2:3, pl.ds(p * 2048, 2048)]
            my2 = masked_ref[3:4, pl.ds(p * 2048, 2048)]
            ma = masked_ref[4:5, pl.ds(p * 2048, 2048)]
            wx = jnp.clip(jnp.minimum(x2c, mx2) - jnp.maximum(x1c, mx1), 0.0)
            wy = jnp.clip(jnp.minimum(y2c, my2) - jnp.maximum(y1c, my1), 0.0)
            inter = wx * wy
            iou = inter / (ac + ma - inter + 1e-9)
            hit = jnp.max(jnp.where(iou > _THR, 1.0, 0.0), axis=1, keepdims=True)
            return jnp.maximum(sup, hit)

        sup0 = lax.fori_loop(0, (b + 15) // 16, chunk,
                             jnp.zeros((128, 1), jnp.float32))

        wx = jnp.clip(jnp.minimum(x2m, x2r) - jnp.maximum(x1m, x1r), 0.0)
        wy = jnp.clip(jnp.minimum(y2m, y2r) - jnp.maximum(y1m, y1r), 0.0)
        inter = wx * wy
        iou = inter / (am + area_r - inter + 1e-9)
        sub = lax.broadcasted_iota(jnp.int32, (128, 128), 0)
        ln2 = lax.broadcasted_iota(jnp.int32, (128, 128), 1)
        m_ref[...] = jnp.where((iou > _THR) & (ln2 < sub), 1.0, 0.0)

        valid_row = ((off + lane) < _PRE).astype(jnp.float32)
        sup0_row = jnp.transpose(jnp.broadcast_to(sup0, (128, 128)))[0:1, :]
        base_row = valid_row * (1.0 - sup0_row)

        def seq(i, kept):
            m_i = m_ref[pl.ds(i, 1), :]
            supi = jnp.max(kept * m_i, axis=1, keepdims=True)
            b_i = jnp.max(jnp.where(lane == i, base_row, 0.0), axis=1,
                          keepdims=True)
            newv = jnp.where(supi > 0.5, 0.0, b_i)
            return jnp.where(lane == i, newv, kept)

        kept = lax.fori_loop(0, 128, seq, jnp.zeros((1, 128), jnp.float32))
        keep_ref[pl.ds(b, 1), :] = kept
        masked_ref[0:1, pl.ds(off, 128)] = jnp.where(kept > 0, x1r, -1e9)
        masked_ref[1:2, pl.ds(off, 128)] = jnp.where(kept > 0, y1r, -1e9)
        masked_ref[2:3, pl.ds(off, 128)] = jnp.where(kept > 0, x2r, -1e9)
        masked_ref[3:4, pl.ds(off, 128)] = jnp.where(kept > 0, y2r, -1e9)
        masked_ref[4:5, pl.ds(off, 128)] = area_r * kept
        return 0

    lax.fori_loop(0, _NB, block, 0)

    keep = keep_ref[...]
    i0 = lax.broadcasted_iota(jnp.int32, (128, 128), 0)
    i1 = lax.broadcasted_iota(jnp.int32, (128, 128), 1)
    u = (i0 <= i1).astype(jnp.float32)
    prefix = lax.dot_general(keep, u, (((1,), (0,)), ((), ())),
                             preferred_element_type=jnp.float32)
    row_tot = prefix[:, 127:128]
    j0 = lax.broadcasted_iota(jnp.int32, (_NB, _NB), 0)
    j1 = lax.broadcasted_iota(jnp.int32, (_NB, _NB), 1)
    lmat = (j1 < j0).astype(jnp.float32)
    offs = lax.dot_general(lmat, row_tot, (((1,), (0,)), ((), ())),
                           preferred_element_type=jnp.float32)
    total = lax.dot_general(jnp.ones((1, _NB), jnp.float32), row_tot,
                            (((1,), (0,)), ((), ())),
                            preferred_element_type=jnp.float32)
    cum = (prefix + offs).astype(jnp.int32)
    tot = jnp.broadcast_to(total, (_NB, 128)).astype(jnp.int32)
    r0 = lax.broadcasted_iota(jnp.int32, (_NB, 128), 0)
    r1 = lax.broadcasted_iota(jnp.int32, (_NB, 128), 1)
    idx = 128 * r0 + r1
    dest = jnp.where(keep > 0, cum - 1, tot + (idx + 1 - cum) - 1)
    dest_ref[...] = jnp.where(idx < _PRE, dest, jnp.int32(99999))


def _select_kernel(dest_ref, vals_ref, out_ref):
    p_row = lax.broadcasted_iota(jnp.int32, (1, 2048), 1)

    def body(c, acc):
        d = dest_ref[pl.ds(c * 1024, 1024), :]
        p = (d == p_row).astype(jnp.float32)
        v = vals_ref[:, pl.ds(c * 1024, 1024)]
        return acc + lax.dot_general(v, p, (((1,), (0,)), ((), ())),
                                     preferred_element_type=jnp.float32,
                                     precision="highest")

    out_ref[...] = lax.fori_loop(0, 10, body, jnp.zeros((8, 2048), jnp.float32))


def kernel(image, feature_map, target, conv_w, conv_b, cls_w, cls_b, box_w, box_b):
    rep = jax.nn.relu(
        lax.conv_general_dilated(feature_map, conv_w, (1, 1), "SAME",
                                 dimension_numbers=("NCHW", "OIHW", "NCHW"))
        + conv_b[None, :, None, None])
    cls = lax.conv_general_dilated(rep, cls_w, (1, 1), "VALID",
                                   dimension_numbers=("NCHW", "OIHW", "NCHW"))
    cls = cls + cls_b[None, :, None, None]
    box = lax.conv_general_dilated(rep, box_w, (1, 1), "VALID",
                                   dimension_numbers=("NCHW", "OIHW", "NCHW"))
    box = box + box_b[None, :, None, None]

    img_h = float(image.shape[-2])
    img_w = float(image.shape[-1])
    h, w = rep.shape[-2], rep.shape[-1]

    logits = jnp.transpose(cls, (0, 2, 3, 1)).reshape(-1)
    box_r = box.reshape(1, 9, 4, h, w)
    box_r = jnp.transpose(box_r, (0, 3, 4, 1, 2)).reshape(-1, 4)
    logits, box_r = lax.optimization_barrier((logits, box_r))
    logits = jnp.pad(logits, (0, _NP - _N), constant_values=-1e9).reshape(1, _NP)
    deltas = jnp.pad(box_r.T, ((0, 0), (0, _NP - _N)))
    anc = jnp.pad(_anchors(image.shape, feature_map.shape).T,
                  ((0, 0), (0, _NP - _N)))

    vals = pl.pallas_call(
        functools.partial(_prep_kernel, img_h, img_w),
        out_shape=jax.ShapeDtypeStruct((8, _NP), jnp.float32),
    )(logits, deltas, anc)

    s_row = vals[0:1, :]
    s_col = vals[0].reshape(_NP, 1)

    rank = pl.pallas_call(
        _rank_kernel,
        out_shape=jax.ShapeDtypeStruct((_NP, 1), jnp.int32),
        grid=(_NP // _RB,),
        in_specs=[pl.BlockSpec((_RB, 1), lambda i: (i, 0)),
                  pl.BlockSpec((1, _NP), lambda i: (0, 0))],
        out_specs=pl.BlockSpec((_RB, 1), lambda i: (i, 0)),
    )(s_col, s_row)

    sorted_vals = pl.pallas_call(
        _scatter_kernel,
        out_shape=jax.ShapeDtypeStruct((8, _NS), jnp.float32),
        grid=(_NS // 512,),
        in_specs=[pl.BlockSpec((_NP, 1), lambda i: (0, 0)),
                  pl.BlockSpec((8, _NP), lambda i: (0, 0))],
        out_specs=pl.BlockSpec((8, 512), lambda i: (0, i)),
    )(rank, vals)

    dest = pl.pallas_call(
        _nms_kernel,
        out_shape=jax.ShapeDtypeStruct((_NB, 128), jnp.int32),
        scratch_shapes=[pltpu.VMEM((8, _NS), jnp.float32),
                        pltpu.VMEM((128, 128), jnp.float32),
                        pltpu.VMEM((_NB, 128), jnp.float32)],
    )(sorted_vals)

    out = pl.pallas_call(
        _select_kernel,
        out_shape=jax.ShapeDtypeStruct((8, 2048), jnp.float32),
        in_specs=[pl.BlockSpec((_NS, 1), lambda: (0, 0)),
                  pl.BlockSpec((8, _NS), lambda: (0, 0))],
        out_specs=pl.BlockSpec((8, 2048), lambda: (0, 0)),
    )(dest.reshape(_NS, 1), sorted_vals)

    proposals = out[1:5, :_POST].T
    scores = out[0, :_POST]
    return proposals, scores
